# baseline (speedup 1.0000x reference)
import functools

import jax
import jax.numpy as jnp
from jax import lax
from jax.experimental import pallas as pl
from jax.experimental.pallas import tpu as pltpu
from jax.experimental.pallas import tpu_sc as plsc

NE = 8192
ED = 32
RT = 128
NW = 32
RPW = NE // NW
CHUNK = 128
NCH = RPW // CHUNK


def _tc_body(z_ref, w_ref, idx_ref, enc_ref):
    zt = z_ref[...]
    w = w_ref[...]
    wsq = jnp.sum(w * w, axis=1, keepdims=True)
    w_aug = jnp.concatenate([w * -2.0, wsq], axis=1)
    z_aug = jnp.concatenate(
        [zt, jnp.ones((RT, 1), jnp.float32)], axis=1)
    d = lax.dot_general(z_aug, w_aug, (((1,), (1,)), ((), ())),
                        preferred_element_type=jnp.float32)
    dmin = jnp.min(d, axis=1, keepdims=True)
    col = lax.broadcasted_iota(jnp.int32, d.shape, 1)
    idx = jnp.min(jnp.where(d == dmin, col, NE), axis=1)
    idx_ref[...] = idx.reshape(1, 1, RT)
    enc_ref[...] = (col == idx[:, None]).astype(jnp.float32)


_tc_call = pl.pallas_call(
    _tc_body,
    grid=(NE // RT,),
    in_specs=[
        pl.BlockSpec((RT, ED), lambda i: (i, 0)),
        pl.BlockSpec((NE, ED), lambda i: (0, 0)),
    ],
    out_specs=[
        pl.BlockSpec((1, 1, RT), lambda i: (i, 0, 0)),
        pl.BlockSpec((RT, NE), lambda i: (i, 0)),
    ],
    out_shape=[
        jax.ShapeDtypeStruct((NE // RT, 1, RT), jnp.int32),
        jax.ShapeDtypeStruct((NE, NE), jnp.float32),
    ],
)


def _sc_body(z_hbm, w_hbm, idx_hbm, qst_hbm, loss_hbm,
             idx_v, rows_v, z_v, qst_v, acc_v, sem):
    c = lax.axis_index("c")
    s = lax.axis_index("s")
    wid = s * 2 + c
    base = wid * RPW
    pltpu.sync_copy(idx_hbm.at[pl.ds(wid * NCH, NCH)], idx_v)
    acc = jnp.zeros((16,), jnp.float32)
    for j in range(NCH):
        row0 = base + j * CHUNK
        pltpu.async_copy(w_hbm.at[idx_v.at[j]], rows_v, sem).wait()
        pltpu.sync_copy(z_hbm.at[pl.ds(row0, CHUNK)], z_v)

        def body(r, a):
            q0 = rows_v[r, pl.ds(0, 16)]
            q1 = rows_v[r, pl.ds(16, 16)]
            z0 = z_v[r, pl.ds(0, 16)]
            z1 = z_v[r, pl.ds(16, 16)]
            d0 = q0 - z0
            d1 = q1 - z1
            qst_v[r, pl.ds(0, 16)] = z0 + d0
            qst_v[r, pl.ds(16, 16)] = z1 + d1
            return a + d0 * d0 + d1 * d1

        acc = lax.fori_loop(0, CHUNK, body, acc)
        pltpu.sync_copy(qst_v, qst_hbm.at[pl.ds(row0, CHUNK)])
    acc_v[...] = acc
    pltpu.sync_copy(acc_v, loss_hbm.at[wid])


@functools.cache
def _sc_quantize():
    mesh = plsc.VectorSubcoreMesh(
        core_axis_name="c", subcore_axis_name="s", num_cores=2)
    return pl.kernel(
        _sc_body,
        mesh=mesh,
        out_type=[
            jax.ShapeDtypeStruct((NE, ED), jnp.float32),
            jax.ShapeDtypeStruct((NW, 16), jnp.float32),
        ],
        scratch_types=[
            pltpu.VMEM((NCH, CHUNK), jnp.int32),
            pltpu.VMEM((CHUNK, 128), jnp.float32),
            pltpu.VMEM((CHUNK, ED), jnp.float32),
            pltpu.VMEM((CHUNK, ED), jnp.float32),
            pltpu.VMEM((16,), jnp.float32),
            pltpu.SemaphoreType.DMA,
        ],
    )


def kernel(z, W):
    z_flat = z.reshape(NE, ED)
    idx3, enc = _tc_call(z_flat, W)
    idx2 = idx3.reshape(NW * NCH, CHUNK)
    W_pad = jnp.pad(W, ((0, 0), (0, 128 - ED)))
    qst_flat, loss_p = _sc_quantize()(z_flat, W_pad, idx2)
    mse = jnp.sum(loss_p) / jnp.float32(NE * ED)
    loss = jnp.float32(1.25) * mse
    return qst_flat.reshape(z.shape), loss, enc


# speedup vs baseline: 1.3423x; 1.3423x over previous
import functools

import jax
import jax.numpy as jnp
from jax import lax
from jax.experimental import pallas as pl
from jax.experimental.pallas import tpu as pltpu
from jax.experimental.pallas import tpu_sc as plsc

NE = 8192
ED = 32
RT = 256
NW = 32
RPW = NE // NW
CHUNK = 128
NCH = RPW // CHUNK


def _tc_body(z_ref, w_ref, idx_ref, enc_ref):
    zt = z_ref[...]
    w = w_ref[...]
    wsq = jnp.sum(w * w, axis=1, keepdims=True)
    w_aug = jnp.concatenate([w * -2.0, wsq], axis=1)
    z_aug = jnp.concatenate(
        [zt, jnp.ones((RT, 1), jnp.float32)], axis=1)
    d = lax.dot_general(z_aug, w_aug, (((1,), (1,)), ((), ())),
                        preferred_element_type=jnp.float32)
    dmin = jnp.min(d, axis=1, keepdims=True)
    col = lax.broadcasted_iota(jnp.int32, d.shape, 1)
    idx = jnp.min(jnp.where(d == dmin, col, NE), axis=1)
    idx_ref[...] = idx.reshape(1, 1, RT)
    enc_ref[...] = (col == idx[:, None]).astype(jnp.float32)


_tc_call = pl.pallas_call(
    _tc_body,
    grid=(NE // RT,),
    in_specs=[
        pl.BlockSpec((RT, ED), lambda i: (i, 0)),
        pl.BlockSpec((NE, ED), lambda i: (0, 0)),
    ],
    out_specs=[
        pl.BlockSpec((1, 1, RT), lambda i: (i, 0, 0)),
        pl.BlockSpec((RT, NE), lambda i: (i, 0)),
    ],
    out_shape=[
        jax.ShapeDtypeStruct((NE // RT, 1, RT), jnp.int32),
        jax.ShapeDtypeStruct((NE, NE), jnp.float32),
    ],
)


def _sc_body(z_hbm, w_hbm, idx_hbm, qst_hbm, loss_hbm,
             idx_v, rows_v, z_v, qst_v, acc_v, sem):
    c = lax.axis_index("c")
    s = lax.axis_index("s")
    wid = s * 2 + c
    base = wid * RPW
    pltpu.sync_copy(idx_hbm.at[pl.ds(wid * NCH, NCH)], idx_v)
    acc = jnp.zeros((16,), jnp.float32)
    for j in range(NCH):
        row0 = base + j * CHUNK
        pltpu.async_copy(w_hbm.at[idx_v.at[j]], rows_v, sem).wait()
        pltpu.sync_copy(z_hbm.at[pl.ds(row0, CHUNK)], z_v)

        def body(r, a):
            q0 = rows_v[r, pl.ds(0, 16)]
            q1 = rows_v[r, pl.ds(16, 16)]
            z0 = z_v[r, pl.ds(0, 16)]
            z1 = z_v[r, pl.ds(16, 16)]
            d0 = q0 - z0
            d1 = q1 - z1
            qst_v[r, pl.ds(0, 16)] = z0 + d0
            qst_v[r, pl.ds(16, 16)] = z1 + d1
            return a + d0 * d0 + d1 * d1

        acc = lax.fori_loop(0, CHUNK, body, acc)
        pltpu.sync_copy(qst_v, qst_hbm.at[pl.ds(row0, CHUNK)])
    acc_v[...] = acc
    pltpu.sync_copy(acc_v, loss_hbm.at[wid])


@functools.cache
def _sc_quantize():
    mesh = plsc.VectorSubcoreMesh(
        core_axis_name="c", subcore_axis_name="s", num_cores=2)
    return pl.kernel(
        _sc_body,
        mesh=mesh,
        out_type=[
            jax.ShapeDtypeStruct((NE, ED), jnp.float32),
            jax.ShapeDtypeStruct((NW, 16), jnp.float32),
        ],
        scratch_types=[
            pltpu.VMEM((NCH, CHUNK), jnp.int32),
            pltpu.VMEM((CHUNK, 128), jnp.float32),
            pltpu.VMEM((CHUNK, ED), jnp.float32),
            pltpu.VMEM((CHUNK, ED), jnp.float32),
            pltpu.VMEM((16,), jnp.float32),
            pltpu.SemaphoreType.DMA,
        ],
    )


def kernel(z, W):
    z_flat = z.reshape(NE, ED)
    idx3, enc = _tc_call(z_flat, W)
    idx2 = idx3.reshape(NW * NCH, CHUNK)
    W_pad = jnp.pad(W, ((0, 0), (0, 128 - ED)))
    qst_flat, loss_p = _sc_quantize()(z_flat, W_pad, idx2)
    mse = jnp.sum(loss_p) / jnp.float32(NE * ED)
    loss = jnp.float32(1.25) * mse
    return qst_flat.reshape(z.shape), loss, enc


# speedup vs baseline: 1.3881x; 1.0341x over previous
import functools

import jax
import jax.numpy as jnp
from jax import lax
from jax.experimental import pallas as pl
from jax.experimental.pallas import tpu as pltpu
from jax.experimental.pallas import tpu_sc as plsc

NE = 8192
ED = 32
RT = 512
NW = 32
RPW = NE // NW
CHUNK = 128
NCH = RPW // CHUNK


def _tc_body(z_ref, w_ref, idx_ref, enc_ref):
    zt = z_ref[...]
    w = w_ref[...]
    wsq = jnp.sum(w * w, axis=1, keepdims=True)
    w_aug = jnp.concatenate([w * -2.0, wsq], axis=1)
    z_aug = jnp.concatenate(
        [zt, jnp.ones((RT, 1), jnp.float32)], axis=1)
    d = lax.dot_general(z_aug, w_aug, (((1,), (1,)), ((), ())),
                        preferred_element_type=jnp.float32)
    dmin = jnp.min(d, axis=1, keepdims=True)
    col = lax.broadcasted_iota(jnp.int32, d.shape, 1)
    idx = jnp.min(jnp.where(d == dmin, col, NE), axis=1)
    idx_ref[...] = idx.reshape(1, 1, RT)
    enc_ref[...] = (col == idx[:, None]).astype(jnp.float32)


_tc_call = pl.pallas_call(
    _tc_body,
    grid=(NE // RT,),
    in_specs=[
        pl.BlockSpec((RT, ED), lambda i: (i, 0)),
        pl.BlockSpec((NE, ED), lambda i: (0, 0)),
    ],
    out_specs=[
        pl.BlockSpec((1, 1, RT), lambda i: (i, 0, 0)),
        pl.BlockSpec((RT, NE), lambda i: (i, 0)),
    ],
    out_shape=[
        jax.ShapeDtypeStruct((NE // RT, 1, RT), jnp.int32),
        jax.ShapeDtypeStruct((NE, NE), jnp.float32),
    ],
)


def _sc_body(z_hbm, w_hbm, idx_hbm, qst_hbm, loss_hbm,
             idx_v, rows_v, z_v, qst_v, acc_v, sem):
    c = lax.axis_index("c")
    s = lax.axis_index("s")
    wid = s * 2 + c
    base = wid * RPW
    pltpu.sync_copy(idx_hbm.at[pl.ds(wid * NCH, NCH)], idx_v)
    acc = jnp.zeros((16,), jnp.float32)
    for j in range(NCH):
        row0 = base + j * CHUNK
        pltpu.async_copy(w_hbm.at[idx_v.at[j]], rows_v, sem).wait()
        pltpu.sync_copy(z_hbm.at[pl.ds(row0, CHUNK)], z_v)

        def body(r, a):
            q0 = rows_v[r, pl.ds(0, 16)]
            q1 = rows_v[r, pl.ds(16, 16)]
            z0 = z_v[r, pl.ds(0, 16)]
            z1 = z_v[r, pl.ds(16, 16)]
            d0 = q0 - z0
            d1 = q1 - z1
            qst_v[r, pl.ds(0, 16)] = z0 + d0
            qst_v[r, pl.ds(16, 16)] = z1 + d1
            return a + d0 * d0 + d1 * d1

        acc = lax.fori_loop(0, CHUNK, body, acc)
        pltpu.sync_copy(qst_v, qst_hbm.at[pl.ds(row0, CHUNK)])
    acc_v[...] = acc
    pltpu.sync_copy(acc_v, loss_hbm.at[wid])


@functools.cache
def _sc_quantize():
    mesh = plsc.VectorSubcoreMesh(
        core_axis_name="c", subcore_axis_name="s", num_cores=2)
    return pl.kernel(
        _sc_body,
        mesh=mesh,
        out_type=[
            jax.ShapeDtypeStruct((NE, ED), jnp.float32),
            jax.ShapeDtypeStruct((NW, 16), jnp.float32),
        ],
        scratch_types=[
            pltpu.VMEM((NCH, CHUNK), jnp.int32),
            pltpu.VMEM((CHUNK, 128), jnp.float32),
            pltpu.VMEM((CHUNK, ED), jnp.float32),
            pltpu.VMEM((CHUNK, ED), jnp.float32),
            pltpu.VMEM((16,), jnp.float32),
            pltpu.SemaphoreType.DMA,
        ],
    )


def kernel(z, W):
    z_flat = z.reshape(NE, ED)
    idx3, enc = _tc_call(z_flat, W)
    idx2 = idx3.reshape(NW * NCH, CHUNK)
    W_pad = jnp.pad(W, ((0, 0), (0, 128 - ED)))
    qst_flat, loss_p = _sc_quantize()(z_flat, W_pad, idx2)
    mse = jnp.sum(loss_p) / jnp.float32(NE * ED)
    loss = jnp.float32(1.25) * mse
    return qst_flat.reshape(z.shape), loss, enc
